# Initial kernel scaffold; baseline (speedup 1.0000x reference)
import functools

import jax
import jax.numpy as jnp
from jax import lax
from jax.experimental import pallas as pl
from jax.experimental.pallas import tpu as pltpu
from jax.experimental.pallas import tpu_sc as plsc

N = 10000
NPAD = 10240
D = 128
E = 320000
B = 64
T_OUT = 10
NC = 2
NS = 16
NW = NC * NS
CH = 80
CB = 25
NCR = E // CH
CR_PER_TILE = NCR // NW
NBLK = CR_PER_TILE // CB
ROWS_PT = NPAD // NW
ROWS_SC = NPAD // NS

_MESH = plsc.VectorSubcoreMesh(
    core_axis_name="c", subcore_axis_name="s", num_cores=NC, num_subcores=NS)


@functools.partial(
    pl.kernel,
    out_type=[
        jax.ShapeDtypeStruct((NC * NPAD, D), jnp.float32),
        jax.ShapeDtypeStruct((NW, NPAD), jnp.float32),
    ],
    mesh=_MESH,
    scratch_types=[
        pltpu.VMEM((NPAD,), jnp.float32),
        pltpu.VMEM((NPAD,), jnp.float32),
        pltpu.VMEM((CB, CH), jnp.int32),
        pltpu.VMEM((CB, CH), jnp.int32),
        pltpu.VMEM((CH, D), jnp.float32),
        pltpu.VMEM((CH,), jnp.float32),
        pltpu.VMEM((NPAD,), jnp.float32),
        pltpu.VMEM((80, D), jnp.float32),
        pltpu.VMEM_SHARED((NPAD, D), jnp.float32),
        pltpu.SemaphoreType.DMA,
    ],
)
def _edge_agg(h_hbm, als_hbm, ald_hbm, src_hbm, dst_hbm, acc_hbm, den_hbm,
              als_v, ald_v, srcb, dstb, rows, eeb, denp, zb, acc_sh, sem):
    c = lax.axis_index("c")
    s = lax.axis_index("s")
    w = s * NC + c

    zvec = jnp.zeros((16,), jnp.float32)

    def _zb_body(i, carry):
        for k in range(D // 16):
            zb[i, pl.ds(k * 16, 16)] = zvec
        return carry
    lax.fori_loop(0, 80, _zb_body, 0)

    def _dp_body(i, carry):
        denp[pl.ds(i * 16, 16)] = zvec
        return carry
    lax.fori_loop(0, NPAD // 16, _dp_body, 0)

    for k in range(ROWS_SC // 80):
        pltpu.sync_copy(zb, acc_sh.at[pl.ds(s * ROWS_SC + k * 80, 80), :])

    pltpu.sync_copy(als_hbm, als_v)
    pltpu.sync_copy(ald_hbm, ald_v)
    plsc.subcore_barrier()

    base_cr = w * CR_PER_TILE
    for jb in range(NBLK):
        pltpu.sync_copy(src_hbm.at[pl.ds(base_cr + jb * CB, CB), :], srcb)
        pltpu.sync_copy(dst_hbm.at[pl.ds(base_cr + jb * CB, CB), :], dstb)

        def _chunk_body(j, carry):
            cp = pltpu.async_copy(h_hbm.at[srcb.at[j]], rows, sem)
            for g in range(CH // 16):
                sv = srcb[j, pl.ds(g * 16, 16)]
                dv = dstb[j, pl.ds(g * 16, 16)]
                e = plsc.load_gather(als_v, [sv]) + plsc.load_gather(ald_v, [dv])
                ee = jnp.exp(jnp.maximum(e, 0.2 * e))
                eeb[pl.ds(g * 16, 16)] = ee
                plsc.addupdate_scatter(denp, [dv], ee)
            cp.wait()

            def _row_body(r, rcarry):
                sc_ = eeb[r]
                for k in range(D // 16):
                    rows[r, pl.ds(k * 16, 16)] = rows[r, pl.ds(k * 16, 16)] * sc_
                return rcarry
            lax.fori_loop(0, CH, _row_body, 0)

            pltpu.sync_copy(rows, acc_sh.at[dstb.at[j]], add=True)
            return carry
        lax.fori_loop(0, CB, _chunk_body, 0)

    pltpu.sync_copy(denp, den_hbm.at[w])
    plsc.subcore_barrier()
    pltpu.sync_copy(acc_sh.at[pl.ds(s * ROWS_SC, ROWS_SC), :],
                    acc_hbm.at[pl.ds(c * NPAD + s * ROWS_SC, ROWS_SC), :])


@functools.partial(
    pl.kernel,
    out_type=jax.ShapeDtypeStruct((NW * B, 3 * D), jnp.float32),
    mesh=_MESH,
    scratch_types=[
        pltpu.VMEM((B + 1, 3 * D), jnp.float32),
        pltpu.VMEM((ROWS_PT,), jnp.int32),
        pltpu.VMEM((64, D), jnp.float32),
        pltpu.VMEM((64, D), jnp.float32),
        pltpu.VMEM((64, D), jnp.float32),
    ],
)
def _pool(o1_hbm, o2_hbm, o3_hbm, batch_hbm, mx_hbm, accm, bbuf, r1, r2, r3):
    c = lax.axis_index("c")
    s = lax.axis_index("s")
    w = s * NC + c

    ninf = jnp.full((16,), -jnp.inf, jnp.float32)

    def _init_body(i, carry):
        for k in range(3 * D // 16):
            accm[i, pl.ds(k * 16, 16)] = ninf
        return carry
    lax.fori_loop(0, B + 1, _init_body, 0)

    pltpu.sync_copy(batch_hbm.at[pl.ds(w * ROWS_PT, ROWS_PT)], bbuf)

    for cc in range(ROWS_PT // 64):
        base = w * ROWS_PT + cc * 64
        pltpu.sync_copy(o1_hbm.at[pl.ds(base, 64), :], r1)
        pltpu.sync_copy(o2_hbm.at[pl.ds(base, 64), :], r2)
        pltpu.sync_copy(o3_hbm.at[pl.ds(base, 64), :], r3)

        def _row_body(r, carry):
            bi = bbuf[cc * 64 + r]
            for k in range(D // 16):
                accm[bi, pl.ds(k * 16, 16)] = jnp.maximum(
                    accm[bi, pl.ds(k * 16, 16)], r1[r, pl.ds(k * 16, 16)])
            for k in range(D // 16):
                accm[bi, pl.ds(D + k * 16, 16)] = jnp.maximum(
                    accm[bi, pl.ds(D + k * 16, 16)], r2[r, pl.ds(k * 16, 16)])
            for k in range(D // 16):
                accm[bi, pl.ds(2 * D + k * 16, 16)] = jnp.maximum(
                    accm[bi, pl.ds(2 * D + k * 16, 16)], r3[r, pl.ds(k * 16, 16)])
            return carry
        lax.fori_loop(0, 64, _row_body, 0)

    pltpu.sync_copy(accm.at[pl.ds(0, B), :], mx_hbm.at[pl.ds(w * B, B), :])


_R = 512
_GRID = NPAD // _R


def _t1_body(x_ref, w_ref, as_ref, ad_ref, h_ref, als_ref, ald_ref):
    h = jnp.dot(x_ref[...], w_ref[...], preferred_element_type=jnp.float32)
    h_ref[...] = h
    als_ref[...] = jnp.sum(h * as_ref[...][None, :], axis=1)
    ald_ref[...] = jnp.sum(h * ad_ref[...][None, :], axis=1)


_t1 = pl.pallas_call(
    _t1_body,
    grid=(_GRID,),
    in_specs=[
        pl.BlockSpec((_R, D), lambda i: (i, 0)),
        pl.BlockSpec((D, D), lambda i: (0, 0)),
        pl.BlockSpec((D,), lambda i: (0,)),
        pl.BlockSpec((D,), lambda i: (0,)),
    ],
    out_specs=[
        pl.BlockSpec((_R, D), lambda i: (i, 0)),
        pl.BlockSpec((_R,), lambda i: (i,)),
        pl.BlockSpec((_R,), lambda i: (i,)),
    ],
    out_shape=[
        jax.ShapeDtypeStruct((NPAD, D), jnp.float32),
        jax.ShapeDtypeStruct((NPAD,), jnp.float32),
        jax.ShapeDtypeStruct((NPAD,), jnp.float32),
    ],
)


def _combine(a0, a1, dn, h_ref, als_ref, ald_ref, b_ref):
    v = als_ref[...] + ald_ref[...]
    eself = jnp.exp(jnp.maximum(v, 0.2 * v))
    den = jnp.sum(dn[...], axis=0) + eself + 1e-16
    h = h_ref[...]
    num = a0[...] + a1[...] + eself[:, None] * h
    return num / den[:, None] + b_ref[...][None, :]


def _t2_body(a0, a1, dn, h_ref, als_ref, ald_ref, b_ref, wn_ref, asn_ref,
             adn_ref, o_ref, hn_ref, alsn_ref, aldn_ref):
    o = _combine(a0, a1, dn, h_ref, als_ref, ald_ref, b_ref)
    o_ref[...] = o
    hn = jnp.dot(o, wn_ref[...], preferred_element_type=jnp.float32)
    hn_ref[...] = hn
    alsn_ref[...] = jnp.sum(hn * asn_ref[...][None, :], axis=1)
    aldn_ref[...] = jnp.sum(hn * adn_ref[...][None, :], axis=1)


_t2 = pl.pallas_call(
    _t2_body,
    grid=(_GRID,),
    in_specs=[
        pl.BlockSpec((_R, D), lambda i: (i, 0)),
        pl.BlockSpec((_R, D), lambda i: (NPAD // _R + i, 0)),
        pl.BlockSpec((NW, _R), lambda i: (0, i)),
        pl.BlockSpec((_R, D), lambda i: (i, 0)),
        pl.BlockSpec((_R,), lambda i: (i,)),
        pl.BlockSpec((_R,), lambda i: (i,)),
        pl.BlockSpec((D,), lambda i: (0,)),
        pl.BlockSpec((D, D), lambda i: (0, 0)),
        pl.BlockSpec((D,), lambda i: (0,)),
        pl.BlockSpec((D,), lambda i: (0,)),
    ],
    out_specs=[
        pl.BlockSpec((_R, D), lambda i: (i, 0)),
        pl.BlockSpec((_R, D), lambda i: (i, 0)),
        pl.BlockSpec((_R,), lambda i: (i,)),
        pl.BlockSpec((_R,), lambda i: (i,)),
    ],
    out_shape=[
        jax.ShapeDtypeStruct((NPAD, D), jnp.float32),
        jax.ShapeDtypeStruct((NPAD, D), jnp.float32),
        jax.ShapeDtypeStruct((NPAD,), jnp.float32),
        jax.ShapeDtypeStruct((NPAD,), jnp.float32),
    ],
)


def _t2l_body(a0, a1, dn, h_ref, als_ref, ald_ref, b_ref, o_ref):
    o_ref[...] = _combine(a0, a1, dn, h_ref, als_ref, ald_ref, b_ref)


_t2l = pl.pallas_call(
    _t2l_body,
    grid=(_GRID,),
    in_specs=[
        pl.BlockSpec((_R, D), lambda i: (i, 0)),
        pl.BlockSpec((_R, D), lambda i: (NPAD // _R + i, 0)),
        pl.BlockSpec((NW, _R), lambda i: (0, i)),
        pl.BlockSpec((_R, D), lambda i: (i, 0)),
        pl.BlockSpec((_R,), lambda i: (i,)),
        pl.BlockSpec((_R,), lambda i: (i,)),
        pl.BlockSpec((D,), lambda i: (0,)),
    ],
    out_specs=pl.BlockSpec((_R, D), lambda i: (i, 0)),
    out_shape=jax.ShapeDtypeStruct((NPAD, D), jnp.float32),
)


def _t3_body(mx_ref, w1_ref, b1_ref, w2_ref, b2_ref, out_ref):
    g = jnp.full((B, 3 * D), -jnp.inf, jnp.float32)
    for i in range(NW):
        g = jnp.maximum(g, mx_ref[pl.ds(i * B, B), :])
    gr = jnp.dot(g, w1_ref[...], preferred_element_type=jnp.float32)
    gr = jnp.maximum(gr + b1_ref[...][None, :], 0.0)
    out_ref[...] = (jnp.dot(gr, w2_ref[...], preferred_element_type=jnp.float32)
                    + b2_ref[...][None, :])


_t3 = pl.pallas_call(
    _t3_body,
    in_specs=[
        pl.BlockSpec((NW * B, 3 * D), lambda: (0, 0)),
        pl.BlockSpec((3 * D, D), lambda: (0, 0)),
        pl.BlockSpec((D,), lambda: (0,)),
        pl.BlockSpec((D, D), lambda: (0, 0)),
        pl.BlockSpec((D,), lambda: (0,)),
    ],
    out_specs=pl.BlockSpec((B, D), lambda: (0, 0)),
    out_shape=jax.ShapeDtypeStruct((B, D), jnp.float32),
)


@jax.jit
def kernel(x, edge_index, batch, W0, a_src0, a_dst0, b0, W1, a_src1, a_dst1,
           b1, W2, a_src2, a_dst2, b2, fc1_W, fc1_b, fc2_W, fc2_b):
    xp = jnp.pad(x, ((0, NPAD - N), (0, 0)))
    batch_p = jnp.concatenate(
        [batch, jnp.full((NPAD - N,), B, jnp.int32)])
    src2 = edge_index[0].reshape(NCR, CH)
    dst2 = edge_index[1].reshape(NCR, CH)
    fc2_Wp = jnp.pad(fc2_W, ((0, 0), (0, D - T_OUT)))
    fc2_bp = jnp.pad(fc2_b, (0, D - T_OUT))

    h1, als1, ald1 = _t1(xp, W0, a_src0, a_dst0)
    acc1, den1 = _edge_agg(h1, als1, ald1, src2, dst2)
    o1, h2, als2, ald2 = _t2(acc1, acc1, den1, h1, als1, ald1, b0,
                             W1, a_src1, a_dst1)
    acc2, den2 = _edge_agg(h2, als2, ald2, src2, dst2)
    o2, h3, als3, ald3 = _t2(acc2, acc2, den2, h2, als2, ald2, b1,
                             W2, a_src2, a_dst2)
    acc3, den3 = _edge_agg(h3, als3, ald3, src2, dst2)
    o3 = _t2l(acc3, acc3, den3, h3, als3, ald3, b2)
    mx = _pool(o1, o2, o3, batch_p)
    out = _t3(mx, fc1_W, fc1_b, fc2_Wp, fc2_bp)
    return out[:, :T_OUT]


# speedup vs baseline: 31.1059x; 31.1059x over previous
import functools

import jax
import jax.numpy as jnp
from jax import lax
from jax.experimental import pallas as pl
from jax.experimental.pallas import tpu as pltpu
from jax.experimental.pallas import tpu_sc as plsc

N = 10000
NPAD = 10240
D = 128
E = 320000
B = 64
T_OUT = 10
NC = 2
NS = 16
NW = NC * NS
CH = 128
CB = 16
NCR = E // CH
CR_PER_TILE = 80
NCR_PAD = NW * CR_PER_TILE
NBLK = CR_PER_TILE // CB
ROWS_PT = NPAD // NW
ROWS_SC = NPAD // NS

def _edge_agg_body(h_hbm, als_hbm, ald_hbm, src_hbm, dst_hbm, acc_hbm, den_hbm,
              srcb, dstb, rows, eeb, denp, ea, eb, zb, acc_sh, sem, sem2):
    c = lax.axis_index("c")
    s = lax.axis_index("s")
    w = s * NC + c

    zvec = jnp.zeros((16,), jnp.float32)

    def _zb_body(i, carry):
        for k in range(D // 16):
            zb[i, pl.ds(k * 16, 16)] = zvec
        return carry
    lax.fori_loop(0, 16, _zb_body, 0)

    def _dp_body(i, carry):
        denp[pl.ds(i * 16, 16)] = zvec
        return carry
    lax.fori_loop(0, NPAD // 16, _dp_body, 0)

    for k in range(ROWS_SC // 16):
        pltpu.sync_copy(zb, acc_sh.at[pl.ds(s * ROWS_SC + k * 16, 16), :])
    plsc.subcore_barrier()

    base_cr = w * CR_PER_TILE
    for jb in range(NBLK):
        pltpu.sync_copy(src_hbm.at[pl.ds(base_cr + jb * CB, CB), :], srcb)
        pltpu.sync_copy(dst_hbm.at[pl.ds(base_cr + jb * CB, CB), :], dstb)

        def _chunk_body(j, carry):
            cr = base_cr + jb * CB + j

            @pl.when(cr < NCR)
            def _():
                cp = pltpu.async_copy(h_hbm.at[srcb.at[j]], rows, sem)
                ca = pltpu.async_copy(als_hbm.at[srcb.at[j]], ea, sem2)
                cb2 = pltpu.async_copy(ald_hbm.at[dstb.at[j]], eb, sem2)
                ca.wait()
                cb2.wait()
                for g in range(CH // 16):
                    dv = dstb[j, pl.ds(g * 16, 16)]
                    e = ea[pl.ds(g * 16, 16)] + eb[pl.ds(g * 16, 16)]
                    ee = jnp.exp(jnp.maximum(e, 0.2 * e))
                    eeb[pl.ds(g * 16, 16)] = ee
                    plsc.addupdate_scatter(denp, [dv], ee)
                cp.wait()

                def _row_body(r, rcarry):
                    sc_ = eeb[pl.ds(r, 16)][0]
                    for k in range(D // 16):
                        rows[r, pl.ds(k * 16, 16)] = (
                            rows[r, pl.ds(k * 16, 16)] * sc_)
                    return rcarry
                lax.fori_loop(0, CH, _row_body, 0)

                pltpu.sync_copy(rows, acc_sh.at[dstb.at[j]], add=True)
            return carry
        lax.fori_loop(0, CB, _chunk_body, 0)

    pltpu.sync_copy(denp, den_hbm.at[pl.ds(w * NPAD, NPAD)])
    plsc.subcore_barrier()
    pltpu.sync_copy(acc_sh.at[pl.ds(s * ROWS_SC, ROWS_SC), :],
                    acc_hbm.at[pl.ds(c * NPAD + s * ROWS_SC, ROWS_SC), :])


def _pool_body(o1_hbm, o2_hbm, o3_hbm, batch_hbm, mx_hbm, accm, bbuf, r1, r2, r3):
    c = lax.axis_index("c")
    s = lax.axis_index("s")
    w = s * NC + c

    ninf = jnp.full((16,), -jnp.inf, jnp.float32)

    def _init_body(i, carry):
        for k in range(3 * D // 16):
            accm[i, pl.ds(k * 16, 16)] = ninf
        return carry
    lax.fori_loop(0, B + 1, _init_body, 0)

    pltpu.sync_copy(batch_hbm.at[pl.ds(w * ROWS_PT, ROWS_PT)],
                    bbuf.at[pl.ds(0, ROWS_PT)])

    for cc in range(ROWS_PT // 64):
        base = w * ROWS_PT + cc * 64
        pltpu.sync_copy(o1_hbm.at[pl.ds(base, 64), :], r1)
        pltpu.sync_copy(o2_hbm.at[pl.ds(base, 64), :], r2)
        pltpu.sync_copy(o3_hbm.at[pl.ds(base, 64), :], r3)

        def _row_body(r, carry):
            bi = bbuf[pl.ds(cc * 64 + r, 16)][0]
            for k in range(D // 16):
                accm[bi, pl.ds(k * 16, 16)] = jnp.maximum(
                    accm[bi, pl.ds(k * 16, 16)], r1[r, pl.ds(k * 16, 16)])
            for k in range(D // 16):
                accm[bi, pl.ds(D + k * 16, 16)] = jnp.maximum(
                    accm[bi, pl.ds(D + k * 16, 16)], r2[r, pl.ds(k * 16, 16)])
            for k in range(D // 16):
                accm[bi, pl.ds(2 * D + k * 16, 16)] = jnp.maximum(
                    accm[bi, pl.ds(2 * D + k * 16, 16)], r3[r, pl.ds(k * 16, 16)])
            return carry
        lax.fori_loop(0, 64, _row_body, 0)

    pltpu.sync_copy(accm.at[pl.ds(0, B), :], mx_hbm.at[pl.ds(w * B, B), :])


@functools.cache
def _get_sc_kernels():
    mesh = plsc.VectorSubcoreMesh(
        core_axis_name="c", subcore_axis_name="s",
        num_cores=NC, num_subcores=NS)
    cparams = pltpu.CompilerParams(needs_layout_passes=False)
    edge_agg = pl.kernel(
        _edge_agg_body,
        out_type=[
            jax.ShapeDtypeStruct((NC * NPAD, D), jnp.float32),
            jax.ShapeDtypeStruct((NW * NPAD,), jnp.float32),
        ],
        mesh=mesh,
        scratch_types=[
            pltpu.VMEM((CB, CH), jnp.int32),
            pltpu.VMEM((CB, CH), jnp.int32),
            pltpu.VMEM((CH, D), jnp.float32),
            pltpu.VMEM((CH + 16,), jnp.float32),
            pltpu.VMEM((NPAD,), jnp.float32),
            pltpu.VMEM((CH,), jnp.float32),
            pltpu.VMEM((CH,), jnp.float32),
            pltpu.VMEM((16, D), jnp.float32),
            pltpu.VMEM_SHARED((NPAD, D), jnp.float32),
            pltpu.SemaphoreType.DMA,
            pltpu.SemaphoreType.DMA,
        ],
        compiler_params=cparams,
    )
    pool = pl.kernel(
        _pool_body,
        out_type=jax.ShapeDtypeStruct((NW * B, 3 * D), jnp.float32),
        mesh=mesh,
        scratch_types=[
            pltpu.VMEM((B + 1, 3 * D), jnp.float32),
            pltpu.VMEM((ROWS_PT + 16,), jnp.int32),
            pltpu.VMEM((64, D), jnp.float32),
            pltpu.VMEM((64, D), jnp.float32),
            pltpu.VMEM((64, D), jnp.float32),
        ],
        compiler_params=cparams,
    )
    return edge_agg, pool


_R = 512
_GRID = NPAD // _R


def _t1_body(x_ref, w_ref, as_ref, ad_ref, h_ref, als_ref, ald_ref):
    h = jnp.dot(x_ref[...], w_ref[...], preferred_element_type=jnp.float32)
    h_ref[...] = h
    als_ref[...] = jnp.sum(h * as_ref[...][None, :], axis=1)
    ald_ref[...] = jnp.sum(h * ad_ref[...][None, :], axis=1)


_t1 = pl.pallas_call(
    _t1_body,
    grid=(_GRID,),
    in_specs=[
        pl.BlockSpec((_R, D), lambda i: (i, 0)),
        pl.BlockSpec((D, D), lambda i: (0, 0)),
        pl.BlockSpec((D,), lambda i: (0,)),
        pl.BlockSpec((D,), lambda i: (0,)),
    ],
    out_specs=[
        pl.BlockSpec((_R, D), lambda i: (i, 0)),
        pl.BlockSpec((_R,), lambda i: (i,)),
        pl.BlockSpec((_R,), lambda i: (i,)),
    ],
    out_shape=[
        jax.ShapeDtypeStruct((NPAD, D), jnp.float32),
        jax.ShapeDtypeStruct((NPAD,), jnp.float32),
        jax.ShapeDtypeStruct((NPAD,), jnp.float32),
    ],
)


def _combine(a0, a1, dn, h_ref, als_ref, ald_ref, b_ref):
    v = als_ref[...] + ald_ref[...]
    eself = jnp.exp(jnp.maximum(v, 0.2 * v))
    den = jnp.sum(dn[...], axis=0) + eself + 1e-16
    h = h_ref[...]
    num = a0[...] + a1[...] + eself[:, None] * h
    return num / den[:, None] + b_ref[...][None, :]


def _t2_body(a0, a1, dn, h_ref, als_ref, ald_ref, b_ref, wn_ref, asn_ref,
             adn_ref, o_ref, hn_ref, alsn_ref, aldn_ref):
    o = _combine(a0, a1, dn, h_ref, als_ref, ald_ref, b_ref)
    o_ref[...] = o
    hn = jnp.dot(o, wn_ref[...], preferred_element_type=jnp.float32)
    hn_ref[...] = hn
    alsn_ref[...] = jnp.sum(hn * asn_ref[...][None, :], axis=1)
    aldn_ref[...] = jnp.sum(hn * adn_ref[...][None, :], axis=1)


_t2 = pl.pallas_call(
    _t2_body,
    grid=(_GRID,),
    in_specs=[
        pl.BlockSpec((_R, D), lambda i: (i, 0)),
        pl.BlockSpec((_R, D), lambda i: (NPAD // _R + i, 0)),
        pl.BlockSpec((NW, _R), lambda i: (0, i)),
        pl.BlockSpec((_R, D), lambda i: (i, 0)),
        pl.BlockSpec((_R,), lambda i: (i,)),
        pl.BlockSpec((_R,), lambda i: (i,)),
        pl.BlockSpec((D,), lambda i: (0,)),
        pl.BlockSpec((D, D), lambda i: (0, 0)),
        pl.BlockSpec((D,), lambda i: (0,)),
        pl.BlockSpec((D,), lambda i: (0,)),
    ],
    out_specs=[
        pl.BlockSpec((_R, D), lambda i: (i, 0)),
        pl.BlockSpec((_R, D), lambda i: (i, 0)),
        pl.BlockSpec((_R,), lambda i: (i,)),
        pl.BlockSpec((_R,), lambda i: (i,)),
    ],
    out_shape=[
        jax.ShapeDtypeStruct((NPAD, D), jnp.float32),
        jax.ShapeDtypeStruct((NPAD, D), jnp.float32),
        jax.ShapeDtypeStruct((NPAD,), jnp.float32),
        jax.ShapeDtypeStruct((NPAD,), jnp.float32),
    ],
)


def _t2l_body(a0, a1, dn, h_ref, als_ref, ald_ref, b_ref, o_ref):
    o_ref[...] = _combine(a0, a1, dn, h_ref, als_ref, ald_ref, b_ref)


_t2l = pl.pallas_call(
    _t2l_body,
    grid=(_GRID,),
    in_specs=[
        pl.BlockSpec((_R, D), lambda i: (i, 0)),
        pl.BlockSpec((_R, D), lambda i: (NPAD // _R + i, 0)),
        pl.BlockSpec((NW, _R), lambda i: (0, i)),
        pl.BlockSpec((_R, D), lambda i: (i, 0)),
        pl.BlockSpec((_R,), lambda i: (i,)),
        pl.BlockSpec((_R,), lambda i: (i,)),
        pl.BlockSpec((D,), lambda i: (0,)),
    ],
    out_specs=pl.BlockSpec((_R, D), lambda i: (i, 0)),
    out_shape=jax.ShapeDtypeStruct((NPAD, D), jnp.float32),
)


def _t3_body(mx_ref, w1_ref, b1_ref, w2_ref, b2_ref, out_ref):
    g = jnp.full((B, 3 * D), -jnp.inf, jnp.float32)
    for i in range(NW):
        g = jnp.maximum(g, mx_ref[pl.ds(i * B, B), :])
    gr = jnp.dot(g, w1_ref[...], preferred_element_type=jnp.float32)
    gr = jnp.maximum(gr + b1_ref[...][None, :], 0.0)
    out_ref[...] = (jnp.dot(gr, w2_ref[...], preferred_element_type=jnp.float32)
                    + b2_ref[...][None, :])


_t3 = pl.pallas_call(
    _t3_body,
    in_specs=[
        pl.BlockSpec((NW * B, 3 * D), lambda: (0, 0)),
        pl.BlockSpec((3 * D, D), lambda: (0, 0)),
        pl.BlockSpec((D,), lambda: (0,)),
        pl.BlockSpec((D, D), lambda: (0, 0)),
        pl.BlockSpec((D,), lambda: (0,)),
    ],
    out_specs=pl.BlockSpec((B, D), lambda: (0, 0)),
    out_shape=jax.ShapeDtypeStruct((B, D), jnp.float32),
)


@jax.jit
def kernel(x, edge_index, batch, W0, a_src0, a_dst0, b0, W1, a_src1, a_dst1,
           b1, W2, a_src2, a_dst2, b2, fc1_W, fc1_b, fc2_W, fc2_b):
    xp = jnp.pad(x, ((0, NPAD - N), (0, 0)))
    batch_p = jnp.concatenate(
        [batch, jnp.full((NPAD - N,), B, jnp.int32)])
    epad = NCR_PAD * CH - E
    src2 = jnp.concatenate(
        [edge_index[0], jnp.zeros((epad,), jnp.int32)]).reshape(NCR_PAD, CH)
    dst2 = jnp.concatenate(
        [edge_index[1], jnp.zeros((epad,), jnp.int32)]).reshape(NCR_PAD, CH)
    fc2_Wp = jnp.pad(fc2_W, ((0, 0), (0, D - T_OUT)))
    fc2_bp = jnp.pad(fc2_b, (0, D - T_OUT))
    _edge_agg, _pool = _get_sc_kernels()

    h1, als1, ald1 = _t1(xp, W0, a_src0, a_dst0)
    acc1, den1 = _edge_agg(h1, als1, ald1, src2, dst2)
    o1, h2, als2, ald2 = _t2(acc1, acc1, den1.reshape(NW, NPAD), h1, als1,
                             ald1, b0, W1, a_src1, a_dst1)
    acc2, den2 = _edge_agg(h2, als2, ald2, src2, dst2)
    o2, h3, als3, ald3 = _t2(acc2, acc2, den2.reshape(NW, NPAD), h2, als2,
                             ald2, b1, W2, a_src2, a_dst2)
    acc3, den3 = _edge_agg(h3, als3, ald3, src2, dst2)
    o3 = _t2l(acc3, acc3, den3.reshape(NW, NPAD), h3, als3, ald3, b2)
    mx = _pool(o1, o2, o3, batch_p)
    out = _t3(mx, fc1_W, fc1_b, fc2_Wp, fc2_bp)
    return out[:, :T_OUT]


# speedup vs baseline: 39.9687x; 1.2849x over previous
import functools

import jax
import jax.numpy as jnp
from jax import lax
from jax.experimental import pallas as pl
from jax.experimental.pallas import tpu as pltpu
from jax.experimental.pallas import tpu_sc as plsc

N = 10000
NPAD = 10240
D = 128
E = 320000
B = 64
T_OUT = 10
NC = 2
NS = 16
NW = NC * NS
CH = 128
CB = 16
NCR = E // CH
CR_PER_TILE = 80
NCR_PAD = NW * CR_PER_TILE
NBLK = CR_PER_TILE // CB
ROWS_PT = NPAD // NW
ROWS_SC = NPAD // NS

def _edge_w_body(als_hbm, ald_hbm, src_hbm, dst_hbm, ee_hbm, den_hbm,
                 als_v, ald_v, srcb, dstb, eeo, denp):
    c = lax.axis_index("c")
    s = lax.axis_index("s")
    w = s * NC + c

    zvec = jnp.zeros((16,), jnp.float32)

    def _dp_body(i, carry):
        denp[pl.ds(i * 16, 16)] = zvec
        return carry
    lax.fori_loop(0, NPAD // 16, _dp_body, 0)

    pltpu.sync_copy(als_hbm, als_v)
    pltpu.sync_copy(ald_hbm, ald_v)

    base_cr = w * CR_PER_TILE
    nt = jnp.minimum(CR_PER_TILE, NCR - w * CR_PER_TILE)
    for jb in range(NBLK):
        m = jnp.clip(nt - jb * CB, 0, CB)

        @pl.when(m > 0)
        def _():
            pltpu.sync_copy(src_hbm.at[pl.ds(base_cr + jb * CB, CB), :], srcb)
            pltpu.sync_copy(dst_hbm.at[pl.ds(base_cr + jb * CB, CB), :], dstb)

            def _chunk_body(j, carry):
                for g in range(CH // 16):
                    sv = srcb[j, pl.ds(g * 16, 16)]
                    dv = dstb[j, pl.ds(g * 16, 16)]
                    e = (plsc.load_gather(als_v, [sv])
                         + plsc.load_gather(ald_v, [dv]))
                    ee = jnp.exp(jnp.maximum(e, 0.2 * e))
                    eeo[pl.ds(j * CH + g * 16, 16)] = ee
                    plsc.addupdate_scatter(denp, [dv], ee)
                return carry
            lax.fori_loop(0, m, _chunk_body, 0)
            pltpu.sync_copy(
                eeo, ee_hbm.at[pl.ds((base_cr + jb * CB) * CH, CB * CH)])

    pltpu.sync_copy(denp, den_hbm.at[pl.ds(w * NPAD, NPAD)])


def _edge_agg_body(h_hbm, ee_hbm, src_hbm, dst_hbm, acc_hbm,
                   srcb, dstb, eebf, rows0, rows1, acc_sh,
                   semg0, semg1, sems0, sems1):
    c = lax.axis_index("c")
    s = lax.axis_index("s")
    w = s * NC + c

    zvec = jnp.zeros((16,), jnp.float32)

    def _zb_body(i, carry):
        for k in range(D // 16):
            rows0[i, pl.ds(k * 16, 16)] = zvec
        return carry
    lax.fori_loop(0, CH, _zb_body, 0)

    for k in range(ROWS_SC // CH):
        pltpu.sync_copy(rows0, acc_sh.at[pl.ds(s * ROWS_SC + k * CH, CH), :])
    plsc.subcore_barrier()

    base_cr = w * CR_PER_TILE
    nt = jnp.minimum(CR_PER_TILE, NCR - w * CR_PER_TILE)

    def _scale(rows_p, jj):
        def _row_body(r, rcarry):
            sc_ = eebf[pl.ds(jj * CH + r, 16)][0]
            for k in range(D // 16):
                rows_p[r, pl.ds(k * 16, 16)] = rows_p[r, pl.ds(k * 16, 16)] * sc_
            return rcarry
        lax.fori_loop(0, CH, _row_body, 0)

    def _g(rows_p, j, sem):
        return pltpu.make_async_copy(h_hbm.at[srcb.at[j]], rows_p, sem)

    def _sc(rows_p, j, sem):
        return pltpu.make_async_copy(rows_p, acc_sh.at[dstb.at[j]], sem)

    for jb in range(NBLK):
        m = jnp.clip(nt - jb * CB, 0, CB)

        @pl.when(m > 0)
        def _():
            pltpu.sync_copy(src_hbm.at[pl.ds(base_cr + jb * CB, CB), :], srcb)
            pltpu.sync_copy(dst_hbm.at[pl.ds(base_cr + jb * CB, CB), :], dstb)
            pltpu.sync_copy(
                ee_hbm.at[pl.ds((base_cr + jb * CB) * CH, CB * CH)],
                eebf.at[pl.ds(0, CB * CH)])
            npair = m // 2
            _g(rows0, 0, semg0).start()

            def _pair_body(i, carry):
                j0 = 2 * i
                j1 = 2 * i + 1
                _g(rows0, j0, semg0).wait()

                @pl.when(i > 0)
                def _():
                    _sc(rows1, j1, sems1).wait()
                _g(rows1, j1, semg1).start()
                _scale(rows0, j0)
                _sc(rows0, j0, sems0).start(add=True)
                _g(rows1, j1, semg1).wait()

                @pl.when(i < npair - 1)
                def _():
                    _sc(rows0, j0, sems0).wait()
                    _g(rows0, j0 + 2, semg0).start()
                _scale(rows1, j1)
                _sc(rows1, j1, sems1).start(add=True)
                return carry
            lax.fori_loop(0, npair, _pair_body, 0)
            _sc(rows0, 0, sems0).wait()
            _sc(rows1, 0, sems1).wait()

    plsc.subcore_barrier()
    pltpu.sync_copy(acc_sh.at[pl.ds(s * ROWS_SC, ROWS_SC), :],
                    acc_hbm.at[pl.ds(c * NPAD + s * ROWS_SC, ROWS_SC), :])


def _pool_body(o1_hbm, o2_hbm, o3_hbm, batch_hbm, mx_hbm, accm, bbuf, r1, r2, r3):
    c = lax.axis_index("c")
    s = lax.axis_index("s")
    w = s * NC + c

    ninf = jnp.full((16,), -jnp.inf, jnp.float32)

    def _init_body(i, carry):
        for k in range(3 * D // 16):
            accm[i, pl.ds(k * 16, 16)] = ninf
        return carry
    lax.fori_loop(0, B + 1, _init_body, 0)

    pltpu.sync_copy(batch_hbm.at[pl.ds(w * ROWS_PT, ROWS_PT)],
                    bbuf.at[pl.ds(0, ROWS_PT)])

    for cc in range(ROWS_PT // 64):
        base = w * ROWS_PT + cc * 64
        pltpu.sync_copy(o1_hbm.at[pl.ds(base, 64), :], r1)
        pltpu.sync_copy(o2_hbm.at[pl.ds(base, 64), :], r2)
        pltpu.sync_copy(o3_hbm.at[pl.ds(base, 64), :], r3)

        def _row_body(r, carry):
            bi = bbuf[pl.ds(cc * 64 + r, 16)][0]
            for k in range(D // 16):
                accm[bi, pl.ds(k * 16, 16)] = jnp.maximum(
                    accm[bi, pl.ds(k * 16, 16)], r1[r, pl.ds(k * 16, 16)])
            for k in range(D // 16):
                accm[bi, pl.ds(D + k * 16, 16)] = jnp.maximum(
                    accm[bi, pl.ds(D + k * 16, 16)], r2[r, pl.ds(k * 16, 16)])
            for k in range(D // 16):
                accm[bi, pl.ds(2 * D + k * 16, 16)] = jnp.maximum(
                    accm[bi, pl.ds(2 * D + k * 16, 16)], r3[r, pl.ds(k * 16, 16)])
            return carry
        lax.fori_loop(0, 64, _row_body, 0)

    pltpu.sync_copy(accm.at[pl.ds(0, B), :], mx_hbm.at[pl.ds(w * B, B), :])


@functools.cache
def _get_sc_kernels():
    mesh = plsc.VectorSubcoreMesh(
        core_axis_name="c", subcore_axis_name="s",
        num_cores=NC, num_subcores=NS)
    cparams = pltpu.CompilerParams(needs_layout_passes=False)
    edge_w = pl.kernel(
        _edge_w_body,
        out_type=[
            jax.ShapeDtypeStruct((NCR_PAD * CH,), jnp.float32),
            jax.ShapeDtypeStruct((NW * NPAD,), jnp.float32),
        ],
        mesh=mesh,
        scratch_types=[
            pltpu.VMEM((NPAD,), jnp.float32),
            pltpu.VMEM((NPAD,), jnp.float32),
            pltpu.VMEM((CB, CH), jnp.int32),
            pltpu.VMEM((CB, CH), jnp.int32),
            pltpu.VMEM((CB * CH,), jnp.float32),
            pltpu.VMEM((NPAD,), jnp.float32),
        ],
        compiler_params=cparams,
    )
    edge_agg = pl.kernel(
        _edge_agg_body,
        out_type=jax.ShapeDtypeStruct((NC * NPAD, D), jnp.float32),
        mesh=mesh,
        scratch_types=[
            pltpu.VMEM((CB, CH), jnp.int32),
            pltpu.VMEM((CB, CH), jnp.int32),
            pltpu.VMEM((CB * CH + 16,), jnp.float32),
            pltpu.VMEM((CH, D), jnp.float32),
            pltpu.VMEM((CH, D), jnp.float32),
            pltpu.VMEM_SHARED((NPAD, D), jnp.float32),
            pltpu.SemaphoreType.DMA,
            pltpu.SemaphoreType.DMA,
            pltpu.SemaphoreType.DMA,
            pltpu.SemaphoreType.DMA,
        ],
        compiler_params=cparams,
    )
    pool = pl.kernel(
        _pool_body,
        out_type=jax.ShapeDtypeStruct((NW * B, 3 * D), jnp.float32),
        mesh=mesh,
        scratch_types=[
            pltpu.VMEM((B + 1, 3 * D), jnp.float32),
            pltpu.VMEM((ROWS_PT + 16,), jnp.int32),
            pltpu.VMEM((64, D), jnp.float32),
            pltpu.VMEM((64, D), jnp.float32),
            pltpu.VMEM((64, D), jnp.float32),
        ],
        compiler_params=cparams,
    )
    return edge_w, edge_agg, pool


_R = 512
_GRID = NPAD // _R


def _t1_body(x_ref, w_ref, as_ref, ad_ref, h_ref, als_ref, ald_ref):
    h = jnp.dot(x_ref[...], w_ref[...], preferred_element_type=jnp.float32)
    h_ref[...] = h
    als_ref[...] = jnp.sum(h * as_ref[...][None, :], axis=1)
    ald_ref[...] = jnp.sum(h * ad_ref[...][None, :], axis=1)


_t1 = pl.pallas_call(
    _t1_body,
    grid=(_GRID,),
    in_specs=[
        pl.BlockSpec((_R, D), lambda i: (i, 0)),
        pl.BlockSpec((D, D), lambda i: (0, 0)),
        pl.BlockSpec((D,), lambda i: (0,)),
        pl.BlockSpec((D,), lambda i: (0,)),
    ],
    out_specs=[
        pl.BlockSpec((_R, D), lambda i: (i, 0)),
        pl.BlockSpec((_R,), lambda i: (i,)),
        pl.BlockSpec((_R,), lambda i: (i,)),
    ],
    out_shape=[
        jax.ShapeDtypeStruct((NPAD, D), jnp.float32),
        jax.ShapeDtypeStruct((NPAD,), jnp.float32),
        jax.ShapeDtypeStruct((NPAD,), jnp.float32),
    ],
)


def _combine(a0, a1, dn, h_ref, als_ref, ald_ref, b_ref):
    v = als_ref[...] + ald_ref[...]
    eself = jnp.exp(jnp.maximum(v, 0.2 * v))
    den = jnp.sum(dn[...], axis=0) + eself + 1e-16
    h = h_ref[...]
    num = a0[...] + a1[...] + eself[:, None] * h
    return num / den[:, None] + b_ref[...][None, :]


def _t2_body(a0, a1, dn, h_ref, als_ref, ald_ref, b_ref, wn_ref, asn_ref,
             adn_ref, o_ref, hn_ref, alsn_ref, aldn_ref):
    o = _combine(a0, a1, dn, h_ref, als_ref, ald_ref, b_ref)
    o_ref[...] = o
    hn = jnp.dot(o, wn_ref[...], preferred_element_type=jnp.float32)
    hn_ref[...] = hn
    alsn_ref[...] = jnp.sum(hn * asn_ref[...][None, :], axis=1)
    aldn_ref[...] = jnp.sum(hn * adn_ref[...][None, :], axis=1)


_t2 = pl.pallas_call(
    _t2_body,
    grid=(_GRID,),
    in_specs=[
        pl.BlockSpec((_R, D), lambda i: (i, 0)),
        pl.BlockSpec((_R, D), lambda i: (NPAD // _R + i, 0)),
        pl.BlockSpec((NW, _R), lambda i: (0, i)),
        pl.BlockSpec((_R, D), lambda i: (i, 0)),
        pl.BlockSpec((_R,), lambda i: (i,)),
        pl.BlockSpec((_R,), lambda i: (i,)),
        pl.BlockSpec((D,), lambda i: (0,)),
        pl.BlockSpec((D, D), lambda i: (0, 0)),
        pl.BlockSpec((D,), lambda i: (0,)),
        pl.BlockSpec((D,), lambda i: (0,)),
    ],
    out_specs=[
        pl.BlockSpec((_R, D), lambda i: (i, 0)),
        pl.BlockSpec((_R, D), lambda i: (i, 0)),
        pl.BlockSpec((_R,), lambda i: (i,)),
        pl.BlockSpec((_R,), lambda i: (i,)),
    ],
    out_shape=[
        jax.ShapeDtypeStruct((NPAD, D), jnp.float32),
        jax.ShapeDtypeStruct((NPAD, D), jnp.float32),
        jax.ShapeDtypeStruct((NPAD,), jnp.float32),
        jax.ShapeDtypeStruct((NPAD,), jnp.float32),
    ],
)


def _t2l_body(a0, a1, dn, h_ref, als_ref, ald_ref, b_ref, o_ref):
    o_ref[...] = _combine(a0, a1, dn, h_ref, als_ref, ald_ref, b_ref)


_t2l = pl.pallas_call(
    _t2l_body,
    grid=(_GRID,),
    in_specs=[
        pl.BlockSpec((_R, D), lambda i: (i, 0)),
        pl.BlockSpec((_R, D), lambda i: (NPAD // _R + i, 0)),
        pl.BlockSpec((NW, _R), lambda i: (0, i)),
        pl.BlockSpec((_R, D), lambda i: (i, 0)),
        pl.BlockSpec((_R,), lambda i: (i,)),
        pl.BlockSpec((_R,), lambda i: (i,)),
        pl.BlockSpec((D,), lambda i: (0,)),
    ],
    out_specs=pl.BlockSpec((_R, D), lambda i: (i, 0)),
    out_shape=jax.ShapeDtypeStruct((NPAD, D), jnp.float32),
)


def _t3_body(mx_ref, w1_ref, b1_ref, w2_ref, b2_ref, out_ref):
    g = jnp.full((B, 3 * D), -jnp.inf, jnp.float32)
    for i in range(NW):
        g = jnp.maximum(g, mx_ref[pl.ds(i * B, B), :])
    gr = jnp.dot(g, w1_ref[...], preferred_element_type=jnp.float32)
    gr = jnp.maximum(gr + b1_ref[...][None, :], 0.0)
    out_ref[...] = (jnp.dot(gr, w2_ref[...], preferred_element_type=jnp.float32)
                    + b2_ref[...][None, :])


_t3 = pl.pallas_call(
    _t3_body,
    in_specs=[
        pl.BlockSpec((NW * B, 3 * D), lambda: (0, 0)),
        pl.BlockSpec((3 * D, D), lambda: (0, 0)),
        pl.BlockSpec((D,), lambda: (0,)),
        pl.BlockSpec((D, D), lambda: (0, 0)),
        pl.BlockSpec((D,), lambda: (0,)),
    ],
    out_specs=pl.BlockSpec((B, D), lambda: (0, 0)),
    out_shape=jax.ShapeDtypeStruct((B, D), jnp.float32),
)


@jax.jit
def kernel(x, edge_index, batch, W0, a_src0, a_dst0, b0, W1, a_src1, a_dst1,
           b1, W2, a_src2, a_dst2, b2, fc1_W, fc1_b, fc2_W, fc2_b):
    xp = jnp.pad(x, ((0, NPAD - N), (0, 0)))
    batch_p = jnp.concatenate(
        [batch, jnp.full((NPAD - N,), B, jnp.int32)])
    epad = NCR_PAD * CH - E
    src2 = jnp.concatenate(
        [edge_index[0], jnp.zeros((epad,), jnp.int32)]).reshape(NCR_PAD, CH)
    dst2 = jnp.concatenate(
        [edge_index[1], jnp.zeros((epad,), jnp.int32)]).reshape(NCR_PAD, CH)
    fc2_Wp = jnp.pad(fc2_W, ((0, 0), (0, D - T_OUT)))
    fc2_bp = jnp.pad(fc2_b, (0, D - T_OUT))
    _edge_w, _edge_agg, _pool = _get_sc_kernels()

    h1, als1, ald1 = _t1(xp, W0, a_src0, a_dst0)
    ee1, den1 = _edge_w(als1, ald1, src2, dst2)
    acc1 = _edge_agg(h1, ee1, src2, dst2)
    o1, h2, als2, ald2 = _t2(acc1, acc1, den1.reshape(NW, NPAD), h1, als1,
                             ald1, b0, W1, a_src1, a_dst1)
    ee2, den2 = _edge_w(als2, ald2, src2, dst2)
    acc2 = _edge_agg(h2, ee2, src2, dst2)
    o2, h3, als3, ald3 = _t2(acc2, acc2, den2.reshape(NW, NPAD), h2, als2,
                             ald2, b1, W2, a_src2, a_dst2)
    ee3, den3 = _edge_w(als3, ald3, src2, dst2)
    acc3 = _edge_agg(h3, ee3, src2, dst2)
    o3 = _t2l(acc3, acc3, den3.reshape(NW, NPAD), h3, als3, ald3, b2)
    mx = _pool(o1, o2, o3, batch_p)
    out = _t3(mx, fc1_W, fc1_b, fc2_Wp, fc2_bp)
    return out[:, :T_OUT]


# speedup vs baseline: 44.9336x; 1.1242x over previous
import functools

import jax
import jax.numpy as jnp
from jax import lax
from jax.experimental import pallas as pl
from jax.experimental.pallas import tpu as pltpu
from jax.experimental.pallas import tpu_sc as plsc

N = 10000
NPAD = 10240
D = 128
E = 320000
B = 64
T_OUT = 10
NC = 2
NS = 16
NW = NC * NS
CH = 128
CB = 16
NCR = E // CH
CR_PER_TILE = 80
NCR_PAD = NW * CR_PER_TILE
NBLK = CR_PER_TILE // CB
ROWS_PT = NPAD // NW
ROWS_SC = NPAD // NS

def _edge_w_body(als_hbm, ald_hbm, src_hbm, dst_hbm, ee_hbm, den_hbm,
                 als_v, ald_v, srcb, dstb, eeo, denp):
    c = lax.axis_index("c")
    s = lax.axis_index("s")
    w = s * NC + c

    zvec = jnp.zeros((16,), jnp.float32)

    def _dp_body(i, carry):
        denp[pl.ds(i * 16, 16)] = zvec
        return carry
    lax.fori_loop(0, NPAD // 16, _dp_body, 0)

    pltpu.sync_copy(als_hbm, als_v)
    pltpu.sync_copy(ald_hbm, ald_v)

    base_cr = w * CR_PER_TILE
    nt = jnp.minimum(CR_PER_TILE, NCR - w * CR_PER_TILE)
    for jb in range(NBLK):
        m = jnp.clip(nt - jb * CB, 0, CB)

        @pl.when(m > 0)
        def _():
            pltpu.sync_copy(src_hbm.at[pl.ds(base_cr + jb * CB, CB), :], srcb)
            pltpu.sync_copy(dst_hbm.at[pl.ds(base_cr + jb * CB, CB), :], dstb)

            def _chunk_body(j, carry):
                for g in range(CH // 16):
                    sv = srcb[j, pl.ds(g * 16, 16)]
                    dv = dstb[j, pl.ds(g * 16, 16)]
                    e = (plsc.load_gather(als_v, [sv])
                         + plsc.load_gather(ald_v, [dv]))
                    ee = jnp.exp(jnp.maximum(e, 0.2 * e))
                    eeo[pl.ds(j * CH + g * 16, 16)] = ee
                    plsc.addupdate_scatter(denp, [dv], ee)
                return carry
            lax.fori_loop(0, m, _chunk_body, 0)
            pltpu.sync_copy(
                eeo, ee_hbm.at[pl.ds((base_cr + jb * CB) * CH, CB * CH)])

    pltpu.sync_copy(denp, den_hbm.at[pl.ds(w * NPAD, NPAD)])


def _edge_agg_body(h_hbm, ee_hbm, src_hbm, dst_hbm, acc_hbm,
                   srcb, dstb, eebf, rows0, rows1, acc_sh,
                   semg0, semg1, sems0, sems1):
    c = lax.axis_index("c")
    s = lax.axis_index("s")
    w = s * NC + c

    zvec = jnp.zeros((16,), jnp.float32)

    def _zb_body(i, carry):
        for k in range(D // 16):
            rows0[i, pl.ds(k * 16, 16)] = zvec
        return carry
    lax.fori_loop(0, CH, _zb_body, 0)

    for k in range(ROWS_SC // CH):
        pltpu.sync_copy(rows0, acc_sh.at[pl.ds(s * ROWS_SC + k * CH, CH), :])
    plsc.subcore_barrier()

    base_cr = w * CR_PER_TILE
    nt = jnp.minimum(CR_PER_TILE, NCR - w * CR_PER_TILE)

    def _scale(rows_p, jj):
        def _grp_body(g, rcarry):
            ev = eebf[pl.ds(jj * CH + g * 16, 16)]
            for l in range(16):
                r = g * 16 + l
                s_ = ev[l]
                for k in range(D // 16):
                    rows_p[r, pl.ds(k * 16, 16)] = (
                        rows_p[r, pl.ds(k * 16, 16)] * s_)
            return rcarry
        lax.fori_loop(0, CH // 16, _grp_body, 0)

    def _g(rows_p, j, sem):
        return pltpu.make_async_copy(h_hbm.at[srcb.at[j]], rows_p, sem)

    def _sc(rows_p, j, sem):
        return pltpu.make_async_copy(rows_p, acc_sh.at[dstb.at[j]], sem)

    for jb in range(NBLK):
        m = jnp.clip(nt - jb * CB, 0, CB)

        @pl.when(m > 0)
        def _():
            pltpu.sync_copy(src_hbm.at[pl.ds(base_cr + jb * CB, CB), :], srcb)
            pltpu.sync_copy(dst_hbm.at[pl.ds(base_cr + jb * CB, CB), :], dstb)
            pltpu.sync_copy(
                ee_hbm.at[pl.ds((base_cr + jb * CB) * CH, CB * CH)],
                eebf.at[pl.ds(0, CB * CH)])
            npair = m // 2
            _g(rows0, 0, semg0).start()

            def _pair_body(i, carry):
                j0 = 2 * i
                j1 = 2 * i + 1
                _g(rows0, j0, semg0).wait()

                @pl.when(i > 0)
                def _():
                    _sc(rows1, j1, sems1).wait()
                _g(rows1, j1, semg1).start()
                _scale(rows0, j0)
                _sc(rows0, j0, sems0).start(add=True)
                _g(rows1, j1, semg1).wait()

                @pl.when(i < npair - 1)
                def _():
                    _sc(rows0, j0, sems0).wait()
                    _g(rows0, j0 + 2, semg0).start()
                _scale(rows1, j1)
                _sc(rows1, j1, sems1).start(add=True)
                return carry
            lax.fori_loop(0, npair, _pair_body, 0)
            _sc(rows0, 0, sems0).wait()
            _sc(rows1, 0, sems1).wait()

    plsc.subcore_barrier()
    pltpu.sync_copy(acc_sh.at[pl.ds(s * ROWS_SC, ROWS_SC), :],
                    acc_hbm.at[pl.ds(c * NPAD + s * ROWS_SC, ROWS_SC), :])


def _pool_body(o1_hbm, o2_hbm, o3_hbm, batch_hbm, mx_hbm, accm, bbuf, r1, r2, r3):
    c = lax.axis_index("c")
    s = lax.axis_index("s")
    w = s * NC + c

    ninf = jnp.full((16,), -jnp.inf, jnp.float32)

    def _init_body(i, carry):
        for k in range(3 * D // 16):
            accm[i, pl.ds(k * 16, 16)] = ninf
        return carry
    lax.fori_loop(0, B + 1, _init_body, 0)

    pltpu.sync_copy(batch_hbm.at[pl.ds(w * ROWS_PT, ROWS_PT)],
                    bbuf.at[pl.ds(0, ROWS_PT)])

    for cc in range(ROWS_PT // 64):
        base = w * ROWS_PT + cc * 64
        pltpu.sync_copy(o1_hbm.at[pl.ds(base, 64), :], r1)
        pltpu.sync_copy(o2_hbm.at[pl.ds(base, 64), :], r2)
        pltpu.sync_copy(o3_hbm.at[pl.ds(base, 64), :], r3)

        def _row_body(r, carry):
            bi = bbuf[pl.ds(cc * 64 + r, 16)][0]
            for k in range(D // 16):
                accm[bi, pl.ds(k * 16, 16)] = jnp.maximum(
                    accm[bi, pl.ds(k * 16, 16)], r1[r, pl.ds(k * 16, 16)])
            for k in range(D // 16):
                accm[bi, pl.ds(D + k * 16, 16)] = jnp.maximum(
                    accm[bi, pl.ds(D + k * 16, 16)], r2[r, pl.ds(k * 16, 16)])
            for k in range(D // 16):
                accm[bi, pl.ds(2 * D + k * 16, 16)] = jnp.maximum(
                    accm[bi, pl.ds(2 * D + k * 16, 16)], r3[r, pl.ds(k * 16, 16)])
            return carry
        lax.fori_loop(0, 64, _row_body, 0)

    pltpu.sync_copy(accm.at[pl.ds(0, B), :], mx_hbm.at[pl.ds(w * B, B), :])


@functools.cache
def _get_sc_kernels():
    mesh = plsc.VectorSubcoreMesh(
        core_axis_name="c", subcore_axis_name="s",
        num_cores=NC, num_subcores=NS)
    cparams = pltpu.CompilerParams(needs_layout_passes=False)
    edge_w = pl.kernel(
        _edge_w_body,
        out_type=[
            jax.ShapeDtypeStruct((NCR_PAD * CH,), jnp.float32),
            jax.ShapeDtypeStruct((NW * NPAD,), jnp.float32),
        ],
        mesh=mesh,
        scratch_types=[
            pltpu.VMEM((NPAD,), jnp.float32),
            pltpu.VMEM((NPAD,), jnp.float32),
            pltpu.VMEM((CB, CH), jnp.int32),
            pltpu.VMEM((CB, CH), jnp.int32),
            pltpu.VMEM((CB * CH,), jnp.float32),
            pltpu.VMEM((NPAD,), jnp.float32),
        ],
        compiler_params=cparams,
    )
    edge_agg = pl.kernel(
        _edge_agg_body,
        out_type=jax.ShapeDtypeStruct((NC * NPAD, D), jnp.float32),
        mesh=mesh,
        scratch_types=[
            pltpu.VMEM((CB, CH), jnp.int32),
            pltpu.VMEM((CB, CH), jnp.int32),
            pltpu.VMEM((CB * CH + 16,), jnp.float32),
            pltpu.VMEM((CH, D), jnp.float32),
            pltpu.VMEM((CH, D), jnp.float32),
            pltpu.VMEM_SHARED((NPAD, D), jnp.float32),
            pltpu.SemaphoreType.DMA,
            pltpu.SemaphoreType.DMA,
            pltpu.SemaphoreType.DMA,
            pltpu.SemaphoreType.DMA,
        ],
        compiler_params=cparams,
    )
    pool = pl.kernel(
        _pool_body,
        out_type=jax.ShapeDtypeStruct((NW * B, 3 * D), jnp.float32),
        mesh=mesh,
        scratch_types=[
            pltpu.VMEM((B + 1, 3 * D), jnp.float32),
            pltpu.VMEM((ROWS_PT + 16,), jnp.int32),
            pltpu.VMEM((64, D), jnp.float32),
            pltpu.VMEM((64, D), jnp.float32),
            pltpu.VMEM((64, D), jnp.float32),
        ],
        compiler_params=cparams,
    )
    return edge_w, edge_agg, pool


_R = 512
_GRID = NPAD // _R


def _t1_body(x_ref, w_ref, as_ref, ad_ref, h_ref, als_ref, ald_ref):
    h = jnp.dot(x_ref[...], w_ref[...], preferred_element_type=jnp.float32)
    h_ref[...] = h
    als_ref[...] = jnp.sum(h * as_ref[...][None, :], axis=1)
    ald_ref[...] = jnp.sum(h * ad_ref[...][None, :], axis=1)


_t1 = pl.pallas_call(
    _t1_body,
    grid=(_GRID,),
    in_specs=[
        pl.BlockSpec((_R, D), lambda i: (i, 0)),
        pl.BlockSpec((D, D), lambda i: (0, 0)),
        pl.BlockSpec((D,), lambda i: (0,)),
        pl.BlockSpec((D,), lambda i: (0,)),
    ],
    out_specs=[
        pl.BlockSpec((_R, D), lambda i: (i, 0)),
        pl.BlockSpec((_R,), lambda i: (i,)),
        pl.BlockSpec((_R,), lambda i: (i,)),
    ],
    out_shape=[
        jax.ShapeDtypeStruct((NPAD, D), jnp.float32),
        jax.ShapeDtypeStruct((NPAD,), jnp.float32),
        jax.ShapeDtypeStruct((NPAD,), jnp.float32),
    ],
)


def _combine(a0, a1, dn, h_ref, als_ref, ald_ref, b_ref):
    v = als_ref[...] + ald_ref[...]
    eself = jnp.exp(jnp.maximum(v, 0.2 * v))
    den = jnp.sum(dn[...], axis=0) + eself + 1e-16
    h = h_ref[...]
    num = a0[...] + a1[...] + eself[:, None] * h
    return num / den[:, None] + b_ref[...][None, :]


def _t2_body(a0, a1, dn, h_ref, als_ref, ald_ref, b_ref, wn_ref, asn_ref,
             adn_ref, o_ref, hn_ref, alsn_ref, aldn_ref):
    o = _combine(a0, a1, dn, h_ref, als_ref, ald_ref, b_ref)
    o_ref[...] = o
    hn = jnp.dot(o, wn_ref[...], preferred_element_type=jnp.float32)
    hn_ref[...] = hn
    alsn_ref[...] = jnp.sum(hn * asn_ref[...][None, :], axis=1)
    aldn_ref[...] = jnp.sum(hn * adn_ref[...][None, :], axis=1)


_t2 = pl.pallas_call(
    _t2_body,
    grid=(_GRID,),
    in_specs=[
        pl.BlockSpec((_R, D), lambda i: (i, 0)),
        pl.BlockSpec((_R, D), lambda i: (NPAD // _R + i, 0)),
        pl.BlockSpec((NW, _R), lambda i: (0, i)),
        pl.BlockSpec((_R, D), lambda i: (i, 0)),
        pl.BlockSpec((_R,), lambda i: (i,)),
        pl.BlockSpec((_R,), lambda i: (i,)),
        pl.BlockSpec((D,), lambda i: (0,)),
        pl.BlockSpec((D, D), lambda i: (0, 0)),
        pl.BlockSpec((D,), lambda i: (0,)),
        pl.BlockSpec((D,), lambda i: (0,)),
    ],
    out_specs=[
        pl.BlockSpec((_R, D), lambda i: (i, 0)),
        pl.BlockSpec((_R, D), lambda i: (i, 0)),
        pl.BlockSpec((_R,), lambda i: (i,)),
        pl.BlockSpec((_R,), lambda i: (i,)),
    ],
    out_shape=[
        jax.ShapeDtypeStruct((NPAD, D), jnp.float32),
        jax.ShapeDtypeStruct((NPAD, D), jnp.float32),
        jax.ShapeDtypeStruct((NPAD,), jnp.float32),
        jax.ShapeDtypeStruct((NPAD,), jnp.float32),
    ],
)


def _t2l_body(a0, a1, dn, h_ref, als_ref, ald_ref, b_ref, o_ref):
    o_ref[...] = _combine(a0, a1, dn, h_ref, als_ref, ald_ref, b_ref)


_t2l = pl.pallas_call(
    _t2l_body,
    grid=(_GRID,),
    in_specs=[
        pl.BlockSpec((_R, D), lambda i: (i, 0)),
        pl.BlockSpec((_R, D), lambda i: (NPAD // _R + i, 0)),
        pl.BlockSpec((NW, _R), lambda i: (0, i)),
        pl.BlockSpec((_R, D), lambda i: (i, 0)),
        pl.BlockSpec((_R,), lambda i: (i,)),
        pl.BlockSpec((_R,), lambda i: (i,)),
        pl.BlockSpec((D,), lambda i: (0,)),
    ],
    out_specs=pl.BlockSpec((_R, D), lambda i: (i, 0)),
    out_shape=jax.ShapeDtypeStruct((NPAD, D), jnp.float32),
)


def _t3_body(mx_ref, w1_ref, b1_ref, w2_ref, b2_ref, out_ref):
    g = jnp.full((B, 3 * D), -jnp.inf, jnp.float32)
    for i in range(NW):
        g = jnp.maximum(g, mx_ref[pl.ds(i * B, B), :])
    gr = jnp.dot(g, w1_ref[...], preferred_element_type=jnp.float32)
    gr = jnp.maximum(gr + b1_ref[...][None, :], 0.0)
    out_ref[...] = (jnp.dot(gr, w2_ref[...], preferred_element_type=jnp.float32)
                    + b2_ref[...][None, :])


_t3 = pl.pallas_call(
    _t3_body,
    in_specs=[
        pl.BlockSpec((NW * B, 3 * D), lambda: (0, 0)),
        pl.BlockSpec((3 * D, D), lambda: (0, 0)),
        pl.BlockSpec((D,), lambda: (0,)),
        pl.BlockSpec((D, D), lambda: (0, 0)),
        pl.BlockSpec((D,), lambda: (0,)),
    ],
    out_specs=pl.BlockSpec((B, D), lambda: (0, 0)),
    out_shape=jax.ShapeDtypeStruct((B, D), jnp.float32),
)


@jax.jit
def kernel(x, edge_index, batch, W0, a_src0, a_dst0, b0, W1, a_src1, a_dst1,
           b1, W2, a_src2, a_dst2, b2, fc1_W, fc1_b, fc2_W, fc2_b):
    xp = jnp.pad(x, ((0, NPAD - N), (0, 0)))
    batch_p = jnp.concatenate(
        [batch, jnp.full((NPAD - N,), B, jnp.int32)])
    epad = NCR_PAD * CH - E
    src2 = jnp.concatenate(
        [edge_index[0], jnp.zeros((epad,), jnp.int32)]).reshape(NCR_PAD, CH)
    dst2 = jnp.concatenate(
        [edge_index[1], jnp.zeros((epad,), jnp.int32)]).reshape(NCR_PAD, CH)
    fc2_Wp = jnp.pad(fc2_W, ((0, 0), (0, D - T_OUT)))
    fc2_bp = jnp.pad(fc2_b, (0, D - T_OUT))
    _edge_w, _edge_agg, _pool = _get_sc_kernels()

    h1, als1, ald1 = _t1(xp, W0, a_src0, a_dst0)
    ee1, den1 = _edge_w(als1, ald1, src2, dst2)
    acc1 = _edge_agg(h1, ee1, src2, dst2)
    o1, h2, als2, ald2 = _t2(acc1, acc1, den1.reshape(NW, NPAD), h1, als1,
                             ald1, b0, W1, a_src1, a_dst1)
    ee2, den2 = _edge_w(als2, ald2, src2, dst2)
    acc2 = _edge_agg(h2, ee2, src2, dst2)
    o2, h3, als3, ald3 = _t2(acc2, acc2, den2.reshape(NW, NPAD), h2, als2,
                             ald2, b1, W2, a_src2, a_dst2)
    ee3, den3 = _edge_w(als3, ald3, src2, dst2)
    acc3 = _edge_agg(h3, ee3, src2, dst2)
    o3 = _t2l(acc3, acc3, den3.reshape(NW, NPAD), h3, als3, ald3, b2)
    mx = _pool(o1, o2, o3, batch_p)
    out = _t3(mx, fc1_W, fc1_b, fc2_Wp, fc2_bp)
    return out[:, :T_OUT]


# speedup vs baseline: 48.7672x; 1.0853x over previous
import functools

import jax
import jax.numpy as jnp
from jax import lax
from jax.experimental import pallas as pl
from jax.experimental.pallas import tpu as pltpu
from jax.experimental.pallas import tpu_sc as plsc

N = 10000
NPAD = 10240
D = 128
E = 320000
B = 64
T_OUT = 10
NC = 2
NS = 16
NW = NC * NS
CH = 64
CB = 32
NCR = E // CH
CR_PER_TILE = 160
NCR_PAD = NW * CR_PER_TILE
NBLK = CR_PER_TILE // CB
ROWS_PT = NPAD // NW
ROWS_SC = NPAD // NS

def _edge_w_body(als_hbm, ald_hbm, src_hbm, dst_hbm, ee_hbm, den_hbm,
                 als_v, ald_v, srcb, dstb, eeo, denp):
    c = lax.axis_index("c")
    s = lax.axis_index("s")
    w = s * NC + c

    zvec = jnp.zeros((16,), jnp.float32)

    def _dp_body(i, carry):
        denp[pl.ds(i * 16, 16)] = zvec
        return carry
    lax.fori_loop(0, NPAD // 16, _dp_body, 0)

    pltpu.sync_copy(als_hbm, als_v)
    pltpu.sync_copy(ald_hbm, ald_v)

    base_cr = w * CR_PER_TILE
    nt = jnp.minimum(CR_PER_TILE, NCR - w * CR_PER_TILE)
    for jb in range(NBLK):
        m = jnp.clip(nt - jb * CB, 0, CB)

        @pl.when(m > 0)
        def _():
            pltpu.sync_copy(src_hbm.at[pl.ds(base_cr + jb * CB, CB), :], srcb)
            pltpu.sync_copy(dst_hbm.at[pl.ds(base_cr + jb * CB, CB), :], dstb)

            def _chunk_body(j, carry):
                for g in range(CH // 16):
                    sv = srcb[j, pl.ds(g * 16, 16)]
                    dv = dstb[j, pl.ds(g * 16, 16)]
                    e = (plsc.load_gather(als_v, [sv])
                         + plsc.load_gather(ald_v, [dv]))
                    ee = jnp.exp(jnp.maximum(e, 0.2 * e))
                    eeo[pl.ds(j * CH + g * 16, 16)] = ee
                    plsc.addupdate_scatter(denp, [dv], ee)
                return carry
            lax.fori_loop(0, m, _chunk_body, 0)
            pltpu.sync_copy(
                eeo, ee_hbm.at[pl.ds((base_cr + jb * CB) * CH, CB * CH)])

    pltpu.sync_copy(denp, den_hbm.at[pl.ds(w * NPAD, NPAD)])


def _edge_agg_body(h_hbm, ee_hbm, src_hbm, dst_hbm, acc_hbm,
                   srcb, dstb, eebf, rows0, rows1, rows2, rows3, acc_sh,
                   sg0, sg1, sg2, sg3, ss0, ss1, ss2, ss3):
    c = lax.axis_index("c")
    s = lax.axis_index("s")
    w = s * NC + c

    rows = [rows0, rows1, rows2, rows3]
    sgs = [sg0, sg1, sg2, sg3]
    sss = [ss0, ss1, ss2, ss3]

    zvec = jnp.zeros((16,), jnp.float32)

    def _zb_body(i, carry):
        for k in range(D // 16):
            rows0[i, pl.ds(k * 16, 16)] = zvec
        return carry
    lax.fori_loop(0, CH, _zb_body, 0)

    for k in range(ROWS_SC // CH):
        pltpu.sync_copy(rows0, acc_sh.at[pl.ds(s * ROWS_SC + k * CH, CH), :])
    plsc.subcore_barrier()

    base_cr = w * CR_PER_TILE
    nt = jnp.minimum(CR_PER_TILE, NCR - w * CR_PER_TILE)

    def _scale(rows_p, jj):
        def _grp_body(g, rcarry):
            ev = eebf[pl.ds(jj * CH + g * 16, 16)]
            for l in range(16):
                r = g * 16 + l
                s_ = ev[l]
                for k in range(D // 16):
                    rows_p[r, pl.ds(k * 16, 16)] = (
                        rows_p[r, pl.ds(k * 16, 16)] * s_)
            return rcarry
        lax.fori_loop(0, CH // 16, _grp_body, 0)

    def _g(p, j):
        return pltpu.make_async_copy(h_hbm.at[srcb.at[j]], rows[p], sgs[p])

    def _sc(p, j):
        return pltpu.make_async_copy(rows[p], acc_sh.at[dstb.at[j]], sss[p])

    def _blk_body(jb, carry):
        off = pl.multiple_of(base_cr + jb * CB, 8)
        m = jnp.clip(nt - jb * CB, 0, CB)

        @pl.when(m > 0)
        def _():
            pltpu.sync_copy(src_hbm.at[pl.ds(off, CB), :], srcb)
            pltpu.sync_copy(dst_hbm.at[pl.ds(off, CB), :], dstb)
            eoff = pl.multiple_of(off * CH, 8)
            pltpu.sync_copy(ee_hbm.at[pl.ds(eoff, CB * CH)],
                            eebf.at[pl.ds(0, CB * CH)])
            for p in range(3):
                _g(p, p).start()

            def _quad_body(i, c2):
                for p in range(4):
                    j = 4 * i + p
                    _g(p, j).wait()
                    _scale(rows[p], j)
                    _sc(p, j).start(add=True)
                    jn = j + 3
                    pn = (p + 3) % 4
                    if p == 0:
                        @pl.when((jn < m) & (i > 0))
                        def _():
                            _sc(pn, jn).wait()

                        @pl.when(jn < m)
                        def _():
                            _g(pn, jn).start()
                    else:
                        @pl.when(jn < m)
                        def _():
                            _sc(pn, jn).wait()
                            _g(pn, jn).start()
                return c2
            lax.fori_loop(0, m // 4, _quad_body, 0)
            for p in range(4):
                _sc(p, 0).wait()
        return carry
    lax.fori_loop(0, NBLK, _blk_body, 0)

    plsc.subcore_barrier()
    pltpu.sync_copy(acc_sh.at[pl.ds(s * ROWS_SC, ROWS_SC), :],
                    acc_hbm.at[pl.ds(c * NPAD + s * ROWS_SC, ROWS_SC), :])


def _pool_body(o1_hbm, o2_hbm, o3_hbm, batch_hbm, mx_hbm, accm, bbuf, r1, r2, r3):
    c = lax.axis_index("c")
    s = lax.axis_index("s")
    w = s * NC + c

    ninf = jnp.full((16,), -jnp.inf, jnp.float32)

    def _init_body(i, carry):
        for k in range(3 * D // 16):
            accm[i, pl.ds(k * 16, 16)] = ninf
        return carry
    lax.fori_loop(0, B + 1, _init_body, 0)

    pltpu.sync_copy(batch_hbm.at[pl.ds(w * ROWS_PT, ROWS_PT)],
                    bbuf.at[pl.ds(0, ROWS_PT)])

    for cc in range(ROWS_PT // 64):
        base = w * ROWS_PT + cc * 64
        pltpu.sync_copy(o1_hbm.at[pl.ds(base, 64), :], r1)
        pltpu.sync_copy(o2_hbm.at[pl.ds(base, 64), :], r2)
        pltpu.sync_copy(o3_hbm.at[pl.ds(base, 64), :], r3)

        def _row_body(r, carry):
            bi = bbuf[pl.ds(cc * 64 + r, 16)][0]
            for k in range(D // 16):
                accm[bi, pl.ds(k * 16, 16)] = jnp.maximum(
                    accm[bi, pl.ds(k * 16, 16)], r1[r, pl.ds(k * 16, 16)])
            for k in range(D // 16):
                accm[bi, pl.ds(D + k * 16, 16)] = jnp.maximum(
                    accm[bi, pl.ds(D + k * 16, 16)], r2[r, pl.ds(k * 16, 16)])
            for k in range(D // 16):
                accm[bi, pl.ds(2 * D + k * 16, 16)] = jnp.maximum(
                    accm[bi, pl.ds(2 * D + k * 16, 16)], r3[r, pl.ds(k * 16, 16)])
            return carry
        lax.fori_loop(0, 64, _row_body, 0)

    pltpu.sync_copy(accm.at[pl.ds(0, B), :], mx_hbm.at[pl.ds(w * B, B), :])


@functools.cache
def _get_sc_kernels():
    mesh = plsc.VectorSubcoreMesh(
        core_axis_name="c", subcore_axis_name="s",
        num_cores=NC, num_subcores=NS)
    cparams = pltpu.CompilerParams(needs_layout_passes=False)
    edge_w = pl.kernel(
        _edge_w_body,
        out_type=[
            jax.ShapeDtypeStruct((NCR_PAD * CH,), jnp.float32),
            jax.ShapeDtypeStruct((NW * NPAD,), jnp.float32),
        ],
        mesh=mesh,
        scratch_types=[
            pltpu.VMEM((NPAD,), jnp.float32),
            pltpu.VMEM((NPAD,), jnp.float32),
            pltpu.VMEM((CB, CH), jnp.int32),
            pltpu.VMEM((CB, CH), jnp.int32),
            pltpu.VMEM((CB * CH,), jnp.float32),
            pltpu.VMEM((NPAD,), jnp.float32),
        ],
        compiler_params=cparams,
    )
    edge_agg = pl.kernel(
        _edge_agg_body,
        out_type=jax.ShapeDtypeStruct((NC * NPAD, D), jnp.float32),
        mesh=mesh,
        scratch_types=(
            [
                pltpu.VMEM((CB, CH), jnp.int32),
                pltpu.VMEM((CB, CH), jnp.int32),
                pltpu.VMEM((CB * CH + 16,), jnp.float32),
            ]
            + [pltpu.VMEM((CH, D), jnp.float32)] * 4
            + [pltpu.VMEM_SHARED((NPAD, D), jnp.float32)]
            + [pltpu.SemaphoreType.DMA] * 8
        ),
        compiler_params=cparams,
    )
    pool = pl.kernel(
        _pool_body,
        out_type=jax.ShapeDtypeStruct((NW * B, 3 * D), jnp.float32),
        mesh=mesh,
        scratch_types=[
            pltpu.VMEM((B + 1, 3 * D), jnp.float32),
            pltpu.VMEM((ROWS_PT + 16,), jnp.int32),
            pltpu.VMEM((64, D), jnp.float32),
            pltpu.VMEM((64, D), jnp.float32),
            pltpu.VMEM((64, D), jnp.float32),
        ],
        compiler_params=cparams,
    )
    return edge_w, edge_agg, pool


_R = 512
_GRID = NPAD // _R


def _t1_body(x_ref, w_ref, as_ref, ad_ref, h_ref, als_ref, ald_ref):
    h = jnp.dot(x_ref[...], w_ref[...], preferred_element_type=jnp.float32)
    h_ref[...] = h
    als_ref[...] = jnp.sum(h * as_ref[...][None, :], axis=1)
    ald_ref[...] = jnp.sum(h * ad_ref[...][None, :], axis=1)


_t1 = pl.pallas_call(
    _t1_body,
    grid=(_GRID,),
    in_specs=[
        pl.BlockSpec((_R, D), lambda i: (i, 0)),
        pl.BlockSpec((D, D), lambda i: (0, 0)),
        pl.BlockSpec((D,), lambda i: (0,)),
        pl.BlockSpec((D,), lambda i: (0,)),
    ],
    out_specs=[
        pl.BlockSpec((_R, D), lambda i: (i, 0)),
        pl.BlockSpec((_R,), lambda i: (i,)),
        pl.BlockSpec((_R,), lambda i: (i,)),
    ],
    out_shape=[
        jax.ShapeDtypeStruct((NPAD, D), jnp.float32),
        jax.ShapeDtypeStruct((NPAD,), jnp.float32),
        jax.ShapeDtypeStruct((NPAD,), jnp.float32),
    ],
)


def _combine(a0, a1, dn, h_ref, als_ref, ald_ref, b_ref):
    v = als_ref[...] + ald_ref[...]
    eself = jnp.exp(jnp.maximum(v, 0.2 * v))
    den = jnp.sum(dn[...], axis=0) + eself + 1e-16
    h = h_ref[...]
    num = a0[...] + a1[...] + eself[:, None] * h
    return num / den[:, None] + b_ref[...][None, :]


def _t2_body(a0, a1, dn, h_ref, als_ref, ald_ref, b_ref, wn_ref, asn_ref,
             adn_ref, o_ref, hn_ref, alsn_ref, aldn_ref):
    o = _combine(a0, a1, dn, h_ref, als_ref, ald_ref, b_ref)
    o_ref[...] = o
    hn = jnp.dot(o, wn_ref[...], preferred_element_type=jnp.float32)
    hn_ref[...] = hn
    alsn_ref[...] = jnp.sum(hn * asn_ref[...][None, :], axis=1)
    aldn_ref[...] = jnp.sum(hn * adn_ref[...][None, :], axis=1)


_t2 = pl.pallas_call(
    _t2_body,
    grid=(_GRID,),
    in_specs=[
        pl.BlockSpec((_R, D), lambda i: (i, 0)),
        pl.BlockSpec((_R, D), lambda i: (NPAD // _R + i, 0)),
        pl.BlockSpec((NW, _R), lambda i: (0, i)),
        pl.BlockSpec((_R, D), lambda i: (i, 0)),
        pl.BlockSpec((_R,), lambda i: (i,)),
        pl.BlockSpec((_R,), lambda i: (i,)),
        pl.BlockSpec((D,), lambda i: (0,)),
        pl.BlockSpec((D, D), lambda i: (0, 0)),
        pl.BlockSpec((D,), lambda i: (0,)),
        pl.BlockSpec((D,), lambda i: (0,)),
    ],
    out_specs=[
        pl.BlockSpec((_R, D), lambda i: (i, 0)),
        pl.BlockSpec((_R, D), lambda i: (i, 0)),
        pl.BlockSpec((_R,), lambda i: (i,)),
        pl.BlockSpec((_R,), lambda i: (i,)),
    ],
    out_shape=[
        jax.ShapeDtypeStruct((NPAD, D), jnp.float32),
        jax.ShapeDtypeStruct((NPAD, D), jnp.float32),
        jax.ShapeDtypeStruct((NPAD,), jnp.float32),
        jax.ShapeDtypeStruct((NPAD,), jnp.float32),
    ],
)


def _t2l_body(a0, a1, dn, h_ref, als_ref, ald_ref, b_ref, o_ref):
    o_ref[...] = _combine(a0, a1, dn, h_ref, als_ref, ald_ref, b_ref)


_t2l = pl.pallas_call(
    _t2l_body,
    grid=(_GRID,),
    in_specs=[
        pl.BlockSpec((_R, D), lambda i: (i, 0)),
        pl.BlockSpec((_R, D), lambda i: (NPAD // _R + i, 0)),
        pl.BlockSpec((NW, _R), lambda i: (0, i)),
        pl.BlockSpec((_R, D), lambda i: (i, 0)),
        pl.BlockSpec((_R,), lambda i: (i,)),
        pl.BlockSpec((_R,), lambda i: (i,)),
        pl.BlockSpec((D,), lambda i: (0,)),
    ],
    out_specs=pl.BlockSpec((_R, D), lambda i: (i, 0)),
    out_shape=jax.ShapeDtypeStruct((NPAD, D), jnp.float32),
)


def _t3_body(mx_ref, w1_ref, b1_ref, w2_ref, b2_ref, out_ref):
    g = jnp.full((B, 3 * D), -jnp.inf, jnp.float32)
    for i in range(NW):
        g = jnp.maximum(g, mx_ref[pl.ds(i * B, B), :])
    gr = jnp.dot(g, w1_ref[...], preferred_element_type=jnp.float32)
    gr = jnp.maximum(gr + b1_ref[...][None, :], 0.0)
    out_ref[...] = (jnp.dot(gr, w2_ref[...], preferred_element_type=jnp.float32)
                    + b2_ref[...][None, :])


_t3 = pl.pallas_call(
    _t3_body,
    in_specs=[
        pl.BlockSpec((NW * B, 3 * D), lambda: (0, 0)),
        pl.BlockSpec((3 * D, D), lambda: (0, 0)),
        pl.BlockSpec((D,), lambda: (0,)),
        pl.BlockSpec((D, D), lambda: (0, 0)),
        pl.BlockSpec((D,), lambda: (0,)),
    ],
    out_specs=pl.BlockSpec((B, D), lambda: (0, 0)),
    out_shape=jax.ShapeDtypeStruct((B, D), jnp.float32),
)


@jax.jit
def kernel(x, edge_index, batch, W0, a_src0, a_dst0, b0, W1, a_src1, a_dst1,
           b1, W2, a_src2, a_dst2, b2, fc1_W, fc1_b, fc2_W, fc2_b):
    xp = jnp.pad(x, ((0, NPAD - N), (0, 0)))
    batch_p = jnp.concatenate(
        [batch, jnp.full((NPAD - N,), B, jnp.int32)])
    epad = NCR_PAD * CH - E
    src2 = jnp.concatenate(
        [edge_index[0], jnp.zeros((epad,), jnp.int32)]).reshape(NCR_PAD, CH)
    dst2 = jnp.concatenate(
        [edge_index[1], jnp.zeros((epad,), jnp.int32)]).reshape(NCR_PAD, CH)
    fc2_Wp = jnp.pad(fc2_W, ((0, 0), (0, D - T_OUT)))
    fc2_bp = jnp.pad(fc2_b, (0, D - T_OUT))
    _edge_w, _edge_agg, _pool = _get_sc_kernels()

    h1, als1, ald1 = _t1(xp, W0, a_src0, a_dst0)
    ee1, den1 = _edge_w(als1, ald1, src2, dst2)
    acc1 = _edge_agg(h1, ee1, src2, dst2)
    o1, h2, als2, ald2 = _t2(acc1, acc1, den1.reshape(NW, NPAD), h1, als1,
                             ald1, b0, W1, a_src1, a_dst1)
    ee2, den2 = _edge_w(als2, ald2, src2, dst2)
    acc2 = _edge_agg(h2, ee2, src2, dst2)
    o2, h3, als3, ald3 = _t2(acc2, acc2, den2.reshape(NW, NPAD), h2, als2,
                             ald2, b1, W2, a_src2, a_dst2)
    ee3, den3 = _edge_w(als3, ald3, src2, dst2)
    acc3 = _edge_agg(h3, ee3, src2, dst2)
    o3 = _t2l(acc3, acc3, den3.reshape(NW, NPAD), h3, als3, ald3, b2)
    mx = _pool(o1, o2, o3, batch_p)
    out = _t3(mx, fc1_W, fc1_b, fc2_Wp, fc2_bp)
    return out[:, :T_OUT]
